# baseline (speedup 1.0000x reference)
import functools

import jax
import jax.numpy as jnp
from jax import lax
from jax.experimental import pallas as pl
from jax.experimental.pallas import tpu as pltpu
from jax.experimental.pallas import tpu_sc as plsc

_NC = 2
_NS = 16
_NW = _NC * _NS


def _sc_convert(table_t, tail_pairs):
    d, v = table_t.shape
    ch = 128
    n_full = v // ch
    rem = v - n_full * ch
    mesh = plsc.VectorSubcoreMesh(core_axis_name="c", subcore_axis_name="s")
    n_loop = n_full // _NW + 1

    @functools.partial(
        pl.kernel,
        out_type=jax.ShapeDtypeStruct((v // 2, 2 * d), jnp.float32),
        mesh=mesh,
        scratch_types=[
            pltpu.VMEM((d, ch), jnp.float32),
            pltpu.VMEM((d, ch), jnp.float32),
            pltpu.VMEM((ch // 2, 2 * d), jnp.float32),
            pltpu.VMEM((ch // 2, 2 * d), jnp.float32),
            pltpu.SemaphoreType.DMA,
            pltpu.SemaphoreType.DMA,
            pltpu.SemaphoreType.DMA,
            pltpu.SemaphoreType.DMA,
        ],
        compiler_params=pltpu.CompilerParams(needs_layout_passes=False),
    )
    def convert_kernel(tab_hbm, tail_hbm, out_hbm, in0, in1, st0, st1,
                       isem0, isem1, osem0, osem1):
        wid = lax.axis_index("s") * _NC + lax.axis_index("c")

        def start_in(c, inb, sem):
            off = pl.multiple_of(c * ch, ch)
            pltpu.async_copy(tab_hbm.at[:, pl.ds(off, ch)], inb, sem)

        def wait_in(inb, sem):
            pltpu.make_async_copy(
                tab_hbm.at[:, pl.ds(0, ch)], inb, sem).wait()

        def start_out(c, st, sem):
            pltpu.async_copy(
                st, out_hbm.at[pl.ds(c * (ch // 2), ch // 2)], sem)

        def wait_out(st, sem):
            pltpu.make_async_copy(
                st, out_hbm.at[pl.ds(0, ch // 2)], sem).wait()

        iota = lax.iota(jnp.int32, 16)
        qcols = []
        for m in range(ch // 16):
            t_loc = m * 16 + iota
            qcols.append((lax.shift_right_logical(t_loc, 1),
                          lax.bitwise_and(t_loc, 1) * d))

        def transpose(inb, st, m_lo=0, m_hi=ch // 16):
            @pl.loop(0, d)
            def _(dd):
                for m in range(m_lo, m_hi):
                    q_vec, cbase = qcols[m]
                    vals = plsc.load_gather(
                        inb, [jnp.zeros((16,), jnp.int32) + dd,
                              m * 16 + iota])
                    plsc.store_scatter(st, [q_vec, cbase + dd], vals)

        c0 = wid

        @pl.when(c0 < n_full)
        def _():
            start_in(c0, in0, isem0)

        @pl.loop(0, n_loop, step=2)
        def _(i):
            c = wid + i * _NW
            cn = c + _NW

            @pl.when(cn < n_full)
            def _():
                start_in(cn, in1, isem1)

            @pl.when(c < n_full)
            def _():
                wait_in(in0, isem0)

                @pl.when(i > 0)
                def _():
                    wait_out(st0, osem0)

                transpose(in0, st0)
                start_out(c, st0, osem0)

            cnn = c + 2 * _NW

            @pl.when(cnn < n_full)
            def _():
                start_in(cnn, in0, isem0)

            @pl.when(cn < n_full)
            def _():
                wait_in(in1, isem1)

                @pl.when(i > 0)
                def _():
                    wait_out(st1, osem1)

                transpose(in1, st1)
                start_out(cn, st1, osem1)

        wait_out(st0, osem0)
        wait_out(st1, osem1)

        if rem:
            @pl.when(wid == 4)
            def _():
                pltpu.sync_copy(tail_hbm, st0.at[pl.ds(0, rem // 2)])
                pltpu.sync_copy(
                    st0.at[pl.ds(0, rem // 2)],
                    out_hbm.at[pl.ds(n_full * (ch // 2), rem // 2)])

    return convert_kernel(table_t, tail_pairs)


def _sc_gather(table_pairs, idx_flat):
    n = idx_flat.shape[0]
    per_w = n // _NW
    ch = 128
    n_ch = per_w // ch
    assert per_w % ch == 0 and n_ch % 2 == 0

    mesh = plsc.VectorSubcoreMesh(core_axis_name="c", subcore_axis_name="s")

    @functools.partial(
        pl.kernel,
        out_type=jax.ShapeDtypeStruct((n // 2, 128), jnp.float32),
        mesh=mesh,
        scratch_types=[
            pltpu.VMEM((per_w,), jnp.int32),
            pltpu.VMEM((ch,), jnp.int32),
            pltpu.VMEM((ch,), jnp.int32),
            pltpu.VMEM((ch, 128), jnp.float32),
            pltpu.VMEM((ch, 128), jnp.float32),
            pltpu.VMEM((ch // 2, 128), jnp.float32),
            pltpu.VMEM((ch // 2, 128), jnp.float32),
            pltpu.SemaphoreType.DMA,
            pltpu.SemaphoreType.DMA,
            pltpu.SemaphoreType.DMA,
            pltpu.SemaphoreType.DMA,
        ],
        compiler_params=pltpu.CompilerParams(
            use_tc_tiling_on_sc=False, needs_layout_passes=False),
    )
    def gather_kernel(table_hbm, idx_hbm, out_hbm, idx_v, idxp0, idxp1,
                      buf0, buf1, stage0, stage1,
                      gsem0, gsem1, osem0, osem1):
        wid = lax.axis_index("s") * _NC + lax.axis_index("c")
        tok_base = wid * per_w
        out_base = wid * (per_w // 2)
        pltpu.sync_copy(idx_hbm.at[pl.ds(tok_base, per_w)], idx_v)

        def prep_idxp(i, slot):
            for m in range(ch // 16):
                v = idx_v[pl.ds(i * ch + m * 16, 16)]
                slot[pl.ds(m * 16, 16)] = lax.shift_right_logical(v, 1)

        def start_gather(slot, buf, sem):
            pltpu.async_copy(table_hbm.at[slot], buf, sem)

        def wait_gather(buf, sem):
            pltpu.make_async_copy(table_hbm.at[idxp0], buf, sem).wait()

        def select(i, buf, stage):
            zeros = jnp.zeros((16,), jnp.int32)

            @pl.loop(0, 8)
            def _(tb):
                qblk = lax.div(tb, 2)
                h = lax.rem(tb, 2)
                q_vec = qblk * 16 + lax.iota(jnp.int32, 16)
                j_vec = 2 * q_vec + h
                toks = plsc.load_gather(idx_v, [i * ch + j_vec])
                srcb = lax.bitwise_and(toks, 1) * 64
                for cc in range(64):
                    vals = plsc.load_gather(buf, [j_vec, srcb + cc])
                    plsc.store_scatter(
                        stage, [q_vec, zeros + (h * 64 + cc)], vals)

        def start_out(i, stage, sem):
            pltpu.async_copy(
                stage, out_hbm.at[pl.ds(out_base + i * (ch // 2), ch // 2)],
                sem)

        def wait_out(stage, sem):
            pltpu.make_async_copy(
                stage, out_hbm.at[pl.ds(out_base, ch // 2)], sem).wait()

        prep_idxp(0, idxp0)
        start_gather(idxp0, buf0, gsem0)

        @pl.loop(0, n_ch, step=2)
        def _(i):
            prep_idxp(i + 1, idxp1)
            start_gather(idxp1, buf1, gsem1)
            wait_gather(buf0, gsem0)

            @pl.when(i > 0)
            def _():
                wait_out(stage0, osem0)

            select(i, buf0, stage0)
            start_out(i, stage0, osem0)

            @pl.when(i + 2 < n_ch)
            def _():
                prep_idxp(i + 2, idxp0)
                start_gather(idxp0, buf0, gsem0)

            wait_gather(buf1, gsem1)

            @pl.when(i > 0)
            def _():
                wait_out(stage1, osem1)

            select(i + 1, buf1, stage1)
            start_out(i + 1, stage1, osem1)

        wait_out(stage0, osem0)
        wait_out(stage1, osem1)

    return gather_kernel(table_pairs, idx_flat)


def _tc_dense(tok2, vis2d, pos_tiled, w_t, gamma, beta, blk):
    n_rows, vdim = vis2d.shape
    d = w_t.shape[1]
    grid = n_rows // blk

    def body(tok_ref, vis_ref, pos_ref, w_ref, g_ref, b_ref, out_ref):
        proj = jnp.dot(vis_ref[...], w_ref[...],
                       preferred_element_type=jnp.float32)
        tok2 = tok_ref[...]
        tok = jnp.stack([tok2[:, :d], tok2[:, d:]], axis=1).reshape(blk, d)
        emb = tok + pos_ref[...] + proj
        mean = jnp.mean(emb, axis=1, keepdims=True)
        cent = emb - mean
        var = jnp.mean(cent * cent, axis=1, keepdims=True)
        normed = cent * lax.rsqrt(var + 1e-6)
        out_ref[...] = normed * g_ref[...] + b_ref[...]

    return pl.pallas_call(
        body,
        grid=(grid,),
        in_specs=[
            pl.BlockSpec((blk // 2, 2 * d), lambda i: (i, 0)),
            pl.BlockSpec((blk, vdim), lambda i: (i, 0)),
            pl.BlockSpec((blk, d), lambda i: (0, 0)),
            pl.BlockSpec((vdim, d), lambda i: (0, 0)),
            pl.BlockSpec((1, d), lambda i: (0, 0)),
            pl.BlockSpec((1, d), lambda i: (0, 0)),
        ],
        out_specs=pl.BlockSpec((blk, d), lambda i: (i, 0)),
        out_shape=jax.ShapeDtypeStruct((n_rows, d), jnp.float32),
    )(tok2, vis2d, pos_tiled, w_t, gamma, beta)


def kernel(seq, visual_features, token_table, pos_table, W_visual,
           ln_gamma, ln_beta):
    b, t = seq.shape
    v, d = token_table.shape
    n = b * t
    idx_flat = seq.reshape(n).astype(jnp.int32)

    rem = v % 128
    tail_pairs = token_table[v - rem:, :].reshape(rem // 2, 2 * d)
    table_pairs = _sc_convert(token_table.T, tail_pairs)
    tok2 = _sc_gather(table_pairs, idx_flat)

    vis2d = visual_features.reshape(n, -1)
    blk = 1600
    pos_tiled = jnp.tile(pos_table[:t], (blk // t, 1))
    out2d = _tc_dense(tok2, vis2d, pos_tiled, W_visual.T,
                      ln_gamma.reshape(1, d), ln_beta.reshape(1, d), blk)
    return out2d.reshape(b, t, d)
